# baseline (speedup 1.0000x reference)
import functools

import jax
import jax.numpy as jnp
from jax import lax
from jax.experimental import pallas as pl
from jax.experimental.pallas import tpu as pltpu
from jax.experimental.pallas import tpu_sc as plsc

D = 128
ED = 16
N_NODES = 10000
N_EDGES = 320000
NP = 10240
NC, NS, L = 2, 16, 16
NW = NC * NS
CHUNK = 96
CPW = 105
E_PAD = NW * CPW * CHUNK
ROWS_PER_TILE = NP // NS
DW = 128


def _gelu16(x):
    u = 1.5957691216057308 * (x + 0.044715 * (x * x * x))
    u = jnp.clip(u, -30.0, 30.0)
    e = jnp.exp(u)
    return x * (e / (e + 1.0))


def _pq_body(nodes_ref, w1a_ref, w1b_ref, t_ref):
    n = nodes_ref[...]
    t_ref[0] = jnp.dot(n, w1a_ref[...], preferred_element_type=jnp.float32)
    t_ref[1] = jnp.dot(n, w1b_ref[...], preferred_element_type=jnp.float32)


def _pq(nodes_p, w1a, w1b):
    blk = 512
    grid = NP // blk
    return pl.pallas_call(
        _pq_body,
        grid=(grid,),
        in_specs=[
            pl.BlockSpec((blk, D), lambda i: (i, 0)),
            pl.BlockSpec((D, D), lambda i: (0, 0)),
            pl.BlockSpec((D, D), lambda i: (0, 0)),
        ],
        out_specs=pl.BlockSpec((2, blk, D), lambda i: (0, i, 0)),
        out_shape=jax.ShapeDtypeStruct((2, NP, D), jnp.float32),
    )(nodes_p, w1a, w1b)


def _epre_body(ef_ref, w1c_ref, b1_ref, e_ref):
    e_ref[...] = (
        jnp.dot(ef_ref[...], w1c_ref[...], preferred_element_type=jnp.float32)
        + b1_ref[...]
    ).astype(jnp.bfloat16)


def _epre(ef_p, w1c, b1):
    blk = 2016
    grid = E_PAD // blk
    return pl.pallas_call(
        _epre_body,
        grid=(grid,),
        in_specs=[
            pl.BlockSpec((blk, ED), lambda i: (i, 0)),
            pl.BlockSpec((ED, D), lambda i: (0, 0)),
            pl.BlockSpec((1, D), lambda i: (0, 0)),
        ],
        out_specs=pl.BlockSpec((blk, DW), lambda i: (i, 0)),
        out_shape=jax.ShapeDtypeStruct((E_PAD, DW), jnp.bfloat16),
    )(ef_p, w1c, b1)


_SC_MESH = plsc.VectorSubcoreMesh(
    core_axis_name="c", subcore_axis_name="s", num_cores=NC, num_subcores=NS
)


@functools.partial(
    pl.kernel,
    out_type=[
        jax.ShapeDtypeStruct((NC, NP, DW), jnp.float32),
        jax.ShapeDtypeStruct((NC, NS, NP), jnp.float32),
    ],
    mesh=_SC_MESH,
    scratch_types=[
        pltpu.VMEM((CHUNK,), jnp.int32),
        pltpu.VMEM((CHUNK,), jnp.int32),
        pltpu.VMEM((CHUNK,), jnp.int32),
        pltpu.VMEM((CHUNK, D), jnp.float32),
        pltpu.VMEM((CHUNK, D), jnp.float32),
        pltpu.VMEM((CHUNK, DW // 2), jnp.int32),
        pltpu.VMEM((NP,), jnp.float32),
        pltpu.VMEM_SHARED((NP, DW), jnp.float32),
        pltpu.SemaphoreType.DMA,
        pltpu.SemaphoreType.DMA,
    ],
    compiler_params=pltpu.CompilerParams(needs_layout_passes=False),
)
def _sc_agg(t_hbm, e_hbm, src_hbm, dstp_hbm, h_out, deg_out,
            src_v, dstp_v, dsc_v, bufp, bufq, bufe, deg_v, h_sh, semp, semq):
    cid = lax.axis_index("c")
    sid = lax.axis_index("s")
    wid = sid * NC + cid

    zero16 = jnp.zeros((16,), jnp.float32)

    def _zero_deg(i, carry):
        deg_v[pl.ds(i * 16, 16)] = zero16
        return carry

    lax.fori_loop(0, NP // 16, _zero_deg, 0)

    def _zero_buf(i, carry):
        for j in range(DW // 16):
            bufp[i, pl.ds(j * 16, 16)] = zero16
        return carry

    lax.fori_loop(0, CHUNK, _zero_buf, 0)

    base_row = sid * ROWS_PER_TILE
    for k in range(ROWS_PER_TILE // CHUNK):
        pltpu.sync_copy(bufp, h_sh.at[pl.ds(base_row + k * CHUNK, CHUNK)])
    if ROWS_PER_TILE % CHUNK:
        pltpu.sync_copy(
            bufp, h_sh.at[pl.ds(base_row + ROWS_PER_TILE - CHUNK, CHUNK)]
        )
    plsc.subcore_barrier()

    npv = jnp.full((16,), NP, jnp.int32)
    ones16 = jnp.full((16,), 1.0, jnp.float32)
    himask = jnp.full((16,), -65536, jnp.int32)

    def _chunk(t, carry):
        base = (wid * CPW + t) * CHUNK
        pltpu.sync_copy(src_hbm.at[pl.ds(base, CHUNK)], src_v)
        pltpu.sync_copy(dstp_hbm.at[pl.ds(base, CHUNK)], dstp_v)
        cp = pltpu.async_copy(t_hbm.at[src_v], bufp, semp)
        cq = pltpu.async_copy(t_hbm.at[dstp_v], bufq, semq)
        pltpu.sync_copy(e_hbm.at[pl.ds(base, CHUNK)], bufe)
        for k in range(CHUNK // 16):
            dsc_v[pl.ds(k * 16, 16)] = dstp_v[pl.ds(k * 16, 16)] - npv
        cp.wait()
        cq.wait()

        def _row(i, c2):
            for j in range(D // 32):
                ew = bufe[i, pl.ds(j * 16, 16)]
                ea = plsc.bitcast(lax.shift_left(ew, 16), jnp.float32)
                eb = plsc.bitcast(ew & himask, jnp.float32)
                sa = pl.ds(j * 32, 16)
                sb = pl.ds(j * 32 + 16, 16)
                xa = bufp[i, sa] + bufq[i, sa] + ea
                xb = bufp[i, sb] + bufq[i, sb] + eb
                bufp[i, sa] = _gelu16(xa)
                bufp[i, sb] = _gelu16(xb)
            return c2

        lax.fori_loop(0, CHUNK, _row, 0)

        for k in range(CHUNK // 16):
            idx16 = dsc_v[pl.ds(k * 16, 16)]
            plsc.addupdate_scatter(deg_v, [idx16], ones16)

        pltpu.sync_copy(bufp, h_sh.at[dsc_v], add=True)
        return carry

    lax.fori_loop(0, CPW, _chunk, 0)
    plsc.subcore_barrier()

    for k in range(ROWS_PER_TILE // CHUNK):
        r = base_row + k * CHUNK
        pltpu.sync_copy(h_sh.at[pl.ds(r, CHUNK)], h_out.at[cid, pl.ds(r, CHUNK)])
    pltpu.sync_copy(deg_v, deg_out.at[cid, sid])


def _post_body(h_ref, deg_ref, nodes_ref, w2_ref, b2_ref, w3a_ref, w3b_ref,
               b3_ref, w4_ref, b4_ref, out_ref):
    h = h_ref[0] + h_ref[1]
    deg = jnp.sum(deg_ref[...], axis=(0, 1))
    agg = (
        jnp.dot(h, w2_ref[...], preferred_element_type=jnp.float32)
        + deg[:, None] * b2_ref[...]
    )
    x = (
        jnp.dot(nodes_ref[...], w3a_ref[...], preferred_element_type=jnp.float32)
        + jnp.dot(agg, w3b_ref[...], preferred_element_type=jnp.float32)
        + b3_ref[...]
    )
    out_ref[...] = (
        jnp.dot(jax.nn.gelu(x), w4_ref[...], preferred_element_type=jnp.float32)
        + b4_ref[...]
    )


def _post(hpart, deg, nodes_p, w2, b2, w3a, w3b, b3, w4, b4):
    blk = 512
    grid = NP // blk
    full = lambda i: (0, 0)
    return pl.pallas_call(
        _post_body,
        grid=(grid,),
        in_specs=[
            pl.BlockSpec((NC, blk, DW), lambda i: (0, i, 0)),
            pl.BlockSpec((NC, NS, blk), lambda i: (0, 0, i)),
            pl.BlockSpec((blk, D), lambda i: (i, 0)),
            pl.BlockSpec((D, D), full),
            pl.BlockSpec((1, D), full),
            pl.BlockSpec((D, D), full),
            pl.BlockSpec((D, D), full),
            pl.BlockSpec((1, D), full),
            pl.BlockSpec((D, D), full),
            pl.BlockSpec((1, D), full),
        ],
        out_specs=pl.BlockSpec((blk, D), lambda i: (i, 0)),
        out_shape=jax.ShapeDtypeStruct((NP, D), jnp.float32),
    )(hpart, deg, nodes_p, w2, b2, w3a, w3b, b3, w4, b4)


_SIG = [32 * j + (m // 2 if m % 2 == 0 else 16 + m // 2)
        for j in range(D // 32) for m in range(32)]


def kernel(node_features, edge_indices, edge_features, W1, b1, W2, b2, W3, b3, W4, b4):
    nodes = node_features[0]
    src = edge_indices[0, :, 0]
    dst = edge_indices[0, :, 1]
    ef = edge_features[0]

    pad_e = E_PAD - N_EDGES
    pad_idx = jnp.full((pad_e,), N_NODES, jnp.int32)
    src_p = jnp.concatenate([src, pad_idx])
    dstp_p = jnp.concatenate([dst, pad_idx]) + NP
    ef_p = jnp.concatenate([ef, jnp.zeros((pad_e, ED), jnp.float32)])
    nodes_p = jnp.concatenate([nodes, jnp.zeros((NP - N_NODES, D), jnp.float32)])

    W1a, W1b, W1c = W1[:D], W1[D:2 * D], W1[2 * D:]
    W3a, W3b = W3[:D], W3[D:]

    sig = jnp.array(_SIG)
    T = _pq(nodes_p, W1a, W1b).reshape(2 * NP, D)
    Epre = _epre(ef_p, W1c[:, sig], b1[sig].reshape(1, D))
    E32 = lax.bitcast_convert_type(Epre.reshape(E_PAD, DW // 2, 2), jnp.int32)
    hpart, deg = _sc_agg(T, E32, src_p, dstp_p)
    out_p = _post(hpart, deg, nodes_p, W2, b2.reshape(1, D), W3a, W3b,
                  b3.reshape(1, D), W4, b4.reshape(1, D))
    return out_p[:N_NODES][None]


# speedup vs baseline: 1.0714x; 1.0714x over previous
import functools

import jax
import jax.numpy as jnp
from jax import lax
from jax.experimental import pallas as pl
from jax.experimental.pallas import tpu as pltpu
from jax.experimental.pallas import tpu_sc as plsc

D = 128
ED = 16
N_NODES = 10000
N_EDGES = 320000
NP = 10240
NC, NS, L = 2, 16, 16
NW = NC * NS
CHUNK = 64
CPW = 157
E_PAD = NW * CPW * CHUNK
ROWS_PER_TILE = NP // NS
DW = 128


def _gelu16(x):
    u = 1.5957691216057308 * (x + 0.044715 * (x * x * x))
    u = jnp.clip(u, -30.0, 30.0)
    e = jnp.exp(u)
    return x * (e / (e + 1.0))


def _pq_body(nodes_ref, w1a_ref, w1b_ref, t_ref):
    n = nodes_ref[...]
    t_ref[0] = jnp.dot(n, w1a_ref[...], preferred_element_type=jnp.float32)
    t_ref[1] = jnp.dot(n, w1b_ref[...], preferred_element_type=jnp.float32)


def _pq(nodes_p, w1a, w1b):
    blk = 512
    grid = NP // blk
    return pl.pallas_call(
        _pq_body,
        grid=(grid,),
        in_specs=[
            pl.BlockSpec((blk, D), lambda i: (i, 0)),
            pl.BlockSpec((D, D), lambda i: (0, 0)),
            pl.BlockSpec((D, D), lambda i: (0, 0)),
        ],
        out_specs=pl.BlockSpec((2, blk, D), lambda i: (0, i, 0)),
        out_shape=jax.ShapeDtypeStruct((2, NP, D), jnp.float32),
    )(nodes_p, w1a, w1b)


def _epre_body(ef_ref, w1c_ref, b1_ref, e_ref):
    e_ref[...] = (
        jnp.dot(ef_ref[...], w1c_ref[...], preferred_element_type=jnp.float32)
        + b1_ref[...]
    ).astype(jnp.bfloat16)


def _epre(ef_p, w1c, b1):
    blk = 2048
    grid = E_PAD // blk
    return pl.pallas_call(
        _epre_body,
        grid=(grid,),
        in_specs=[
            pl.BlockSpec((blk, ED), lambda i: (i, 0)),
            pl.BlockSpec((ED, D), lambda i: (0, 0)),
            pl.BlockSpec((1, D), lambda i: (0, 0)),
        ],
        out_specs=pl.BlockSpec((blk, DW), lambda i: (i, 0)),
        out_shape=jax.ShapeDtypeStruct((E_PAD, DW), jnp.bfloat16),
    )(ef_p, w1c, b1)


_SC_MESH = plsc.VectorSubcoreMesh(
    core_axis_name="c", subcore_axis_name="s", num_cores=NC, num_subcores=NS
)


@functools.partial(
    pl.kernel,
    out_type=[
        jax.ShapeDtypeStruct((NC, NP, DW), jnp.float32),
        jax.ShapeDtypeStruct((NC, NS, NP), jnp.float32),
    ],
    mesh=_SC_MESH,
    scratch_types=[
        pltpu.VMEM((2 * CHUNK,), jnp.int32),
        pltpu.VMEM((CHUNK,), jnp.int32),
        pltpu.VMEM((2 * CHUNK, D), jnp.float32),
        pltpu.VMEM((CHUNK, DW // 2), jnp.int32),
        pltpu.VMEM((CHUNK, DW), jnp.float32),
        pltpu.VMEM((NP,), jnp.float32),
        pltpu.VMEM_SHARED((NP, DW), jnp.float32),
        pltpu.SemaphoreType.DMA,
    ],
    compiler_params=pltpu.CompilerParams(needs_layout_passes=False),
)
def _sc_agg(t_hbm, e_hbm, idx_hbm, h_out, deg_out,
            idx_v, dsc_v, bufpq, bufe, bufh, deg_v, h_sh, semg):
    cid = lax.axis_index("c")
    sid = lax.axis_index("s")
    wid = sid * NC + cid

    zero16 = jnp.zeros((16,), jnp.float32)

    def _zero_deg(i, carry):
        deg_v[pl.ds(i * 16, 16)] = zero16
        return carry

    lax.fori_loop(0, NP // 16, _zero_deg, 0)

    def _zero_buf(i, carry):
        for j in range(DW // 16):
            bufh[i, pl.ds(j * 16, 16)] = zero16
        return carry

    lax.fori_loop(0, CHUNK, _zero_buf, 0)

    base_row = sid * ROWS_PER_TILE
    for k in range(ROWS_PER_TILE // CHUNK):
        pltpu.sync_copy(bufh, h_sh.at[pl.ds(base_row + k * CHUNK, CHUNK)])
    plsc.subcore_barrier()

    npv = jnp.full((16,), NP, jnp.int32)
    ones16 = jnp.full((16,), 1.0, jnp.float32)

    def _chunk(t, carry):
        base = (wid * CPW + t) * CHUNK
        pltpu.sync_copy(idx_hbm.at[pl.ds(2 * base, 2 * CHUNK)], idx_v)
        cg = pltpu.async_copy(t_hbm.at[idx_v], bufpq, semg)
        pltpu.sync_copy(e_hbm.at[pl.ds(base, CHUNK)], bufe)
        for k in range(CHUNK // 16):
            dsc_v[pl.ds(k * 16, 16)] = idx_v[pl.ds(CHUNK + k * 16, 16)] - npv
        cg.wait()

        himask = jnp.full((16,), -65536, jnp.int32)

        def _row(i, c2):
            for j in range(D // 32):
                ew = bufe[i, pl.ds(j * 16, 16)]
                ea = plsc.bitcast(lax.shift_left(ew, 16), jnp.float32)
                eb = plsc.bitcast(ew & himask, jnp.float32)
                sa = pl.ds(j * 32, 16)
                sb = pl.ds(j * 32 + 16, 16)
                xa = bufpq[i, sa] + bufpq[CHUNK + i, sa] + ea
                xb = bufpq[i, sb] + bufpq[CHUNK + i, sb] + eb
                bufh[i, sa] = _gelu16(xa)
                bufh[i, sb] = _gelu16(xb)
            return c2

        lax.fori_loop(0, CHUNK, _row, 0)

        for k in range(CHUNK // 16):
            idx16 = dsc_v[pl.ds(k * 16, 16)]
            plsc.addupdate_scatter(deg_v, [idx16], ones16)

        pltpu.sync_copy(bufh, h_sh.at[dsc_v], add=True)
        return carry

    lax.fori_loop(0, CPW, _chunk, 0)
    plsc.subcore_barrier()

    for k in range(ROWS_PER_TILE // CHUNK):
        r = base_row + k * CHUNK
        pltpu.sync_copy(h_sh.at[pl.ds(r, CHUNK)], h_out.at[cid, pl.ds(r, CHUNK)])
    pltpu.sync_copy(deg_v, deg_out.at[cid, sid])


def _post_body(h_ref, deg_ref, nodes_ref, w2_ref, b2_ref, w3a_ref, w3b_ref,
               b3_ref, w4_ref, b4_ref, out_ref):
    h = h_ref[0] + h_ref[1]
    deg = jnp.sum(deg_ref[...], axis=(0, 1))
    agg = (
        jnp.dot(h, w2_ref[...], preferred_element_type=jnp.float32)
        + deg[:, None] * b2_ref[...]
    )
    x = (
        jnp.dot(nodes_ref[...], w3a_ref[...], preferred_element_type=jnp.float32)
        + jnp.dot(agg, w3b_ref[...], preferred_element_type=jnp.float32)
        + b3_ref[...]
    )
    out_ref[...] = (
        jnp.dot(jax.nn.gelu(x), w4_ref[...], preferred_element_type=jnp.float32)
        + b4_ref[...]
    )


def _post(hpart, deg, nodes_p, w2, b2, w3a, w3b, b3, w4, b4):
    blk = 512
    grid = NP // blk
    full = lambda i: (0, 0)
    return pl.pallas_call(
        _post_body,
        grid=(grid,),
        in_specs=[
            pl.BlockSpec((NC, blk, DW), lambda i: (0, i, 0)),
            pl.BlockSpec((NC, NS, blk), lambda i: (0, 0, i)),
            pl.BlockSpec((blk, D), lambda i: (i, 0)),
            pl.BlockSpec((D, D), full),
            pl.BlockSpec((1, D), full),
            pl.BlockSpec((D, D), full),
            pl.BlockSpec((D, D), full),
            pl.BlockSpec((1, D), full),
            pl.BlockSpec((D, D), full),
            pl.BlockSpec((1, D), full),
        ],
        out_specs=pl.BlockSpec((blk, D), lambda i: (i, 0)),
        out_shape=jax.ShapeDtypeStruct((NP, D), jnp.float32),
    )(hpart, deg, nodes_p, w2, b2, w3a, w3b, b3, w4, b4)


_SIG = [32 * j + (m // 2 if m % 2 == 0 else 16 + m // 2)
        for j in range(D // 32) for m in range(32)]


def kernel(node_features, edge_indices, edge_features, W1, b1, W2, b2, W3, b3, W4, b4):
    nodes = node_features[0]
    src = edge_indices[0, :, 0]
    dst = edge_indices[0, :, 1]
    ef = edge_features[0]

    pad_e = E_PAD - N_EDGES
    pad_idx = jnp.full((pad_e,), N_NODES, jnp.int32)
    src_p = jnp.concatenate([src, pad_idx])
    dst_p = jnp.concatenate([dst, pad_idx])
    idxcat = jnp.concatenate(
        [src_p.reshape(-1, CHUNK), dst_p.reshape(-1, CHUNK) + NP], axis=1
    ).reshape(-1)
    ef_p = jnp.concatenate([ef, jnp.zeros((pad_e, ED), jnp.float32)])
    nodes_p = jnp.concatenate([nodes, jnp.zeros((NP - N_NODES, D), jnp.float32)])

    W1a, W1b, W1c = W1[:D], W1[D:2 * D], W1[2 * D:]
    W3a, W3b = W3[:D], W3[D:]

    sig = jnp.array(_SIG)
    T = _pq(nodes_p, W1a, W1b).reshape(2 * NP, D)
    Epre = _epre(ef_p, W1c[:, sig], b1[sig].reshape(1, D))
    E32 = lax.bitcast_convert_type(Epre.reshape(E_PAD, DW // 2, 2), jnp.int32)
    hpart, deg = _sc_agg(T, E32, idxcat)
    out_p = _post(hpart, deg, nodes_p, W2, b2.reshape(1, D), W3a, W3b,
                  b3.reshape(1, D), W4, b4.reshape(1, D))
    return out_p[:N_NODES][None]


# speedup vs baseline: 1.8547x; 1.7310x over previous
import functools

import jax
import jax.numpy as jnp
from jax import lax
from jax.experimental import pallas as pl
from jax.experimental.pallas import tpu as pltpu
from jax.experimental.pallas import tpu_sc as plsc

D = 128
ED = 16
N_NODES = 10000
N_EDGES = 320000
NP = 10240
NC, NS, L = 2, 16, 16
NW = NC * NS
CHUNK = 64
CPW = 157
E_PAD = NW * CPW * CHUNK
ROWS_PER_TILE = NP // NS
DW = 128


def _gelu16(x):
    u = 1.5957691216057308 * (x + 0.044715 * (x * x * x))
    u = jnp.clip(u, -30.0, 30.0)
    e = jnp.exp(u)
    return x * (e / (e + 1.0))


def _pq_body(nodes_ref, w1a_ref, w1b_ref, t_ref):
    n = nodes_ref[...]
    t_ref[0] = jnp.dot(n, w1a_ref[...], preferred_element_type=jnp.float32)
    t_ref[1] = jnp.dot(n, w1b_ref[...], preferred_element_type=jnp.float32)


def _pq(nodes_p, w1a, w1b):
    blk = 512
    grid = NP // blk
    return pl.pallas_call(
        _pq_body,
        grid=(grid,),
        in_specs=[
            pl.BlockSpec((blk, D), lambda i: (i, 0)),
            pl.BlockSpec((D, D), lambda i: (0, 0)),
            pl.BlockSpec((D, D), lambda i: (0, 0)),
        ],
        out_specs=pl.BlockSpec((2, blk, D), lambda i: (0, i, 0)),
        out_shape=jax.ShapeDtypeStruct((2, NP, D), jnp.float32),
    )(nodes_p, w1a, w1b)


def _epre_body(ef_ref, w1c_ref, b1_ref, e_ref):
    e_ref[...] = (
        jnp.dot(ef_ref[...], w1c_ref[...], preferred_element_type=jnp.float32)
        + b1_ref[...]
    )


def _epre(ef_p, w1c, b1):
    blk = 2048
    grid = E_PAD // blk
    return pl.pallas_call(
        _epre_body,
        grid=(grid,),
        in_specs=[
            pl.BlockSpec((blk, ED), lambda i: (i, 0)),
            pl.BlockSpec((ED, D), lambda i: (0, 0)),
            pl.BlockSpec((1, D), lambda i: (0, 0)),
        ],
        out_specs=pl.BlockSpec((blk, DW), lambda i: (i, 0)),
        out_shape=jax.ShapeDtypeStruct((E_PAD, DW), jnp.float32),
    )(ef_p, w1c, b1)


_SC_MESH = plsc.VectorSubcoreMesh(
    core_axis_name="c", subcore_axis_name="s", num_cores=NC, num_subcores=NS
)


@functools.partial(
    pl.kernel,
    out_type=[
        jax.ShapeDtypeStruct((NC, NP, DW), jnp.float32),
        jax.ShapeDtypeStruct((NC, NS, NP), jnp.float32),
    ],
    mesh=_SC_MESH,
    scratch_types=[
        pltpu.VMEM((2 * CHUNK,), jnp.int32),
        pltpu.VMEM((CHUNK,), jnp.int32),
        pltpu.VMEM((2 * CHUNK, D), jnp.float32),
        pltpu.VMEM((CHUNK, DW), jnp.float32),
        pltpu.VMEM((NP,), jnp.float32),
        pltpu.VMEM_SHARED((NP, DW), jnp.float32),
        pltpu.SemaphoreType.DMA,
    ],
    compiler_params=pltpu.CompilerParams(needs_layout_passes=False),
)
def _sc_agg(t_hbm, e_hbm, idx_hbm, h_out, deg_out,
            idx_v, dsc_v, bufpq, bufe, deg_v, h_sh, semg):
    cid = lax.axis_index("c")
    sid = lax.axis_index("s")
    wid = sid * NC + cid

    zero16 = jnp.zeros((16,), jnp.float32)

    def _zero_deg(i, carry):
        deg_v[pl.ds(i * 16, 16)] = zero16
        return carry

    lax.fori_loop(0, NP // 16, _zero_deg, 0)

    def _zero_buf(i, carry):
        for j in range(DW // 16):
            bufe[i, pl.ds(j * 16, 16)] = zero16
        return carry

    lax.fori_loop(0, CHUNK, _zero_buf, 0)

    base_row = sid * ROWS_PER_TILE
    for k in range(ROWS_PER_TILE // CHUNK):
        pltpu.sync_copy(bufe, h_sh.at[pl.ds(base_row + k * CHUNK, CHUNK)])
    plsc.subcore_barrier()

    npv = jnp.full((16,), NP, jnp.int32)
    ones16 = jnp.full((16,), 1.0, jnp.float32)

    def _chunk(t, carry):
        base = (wid * CPW + t) * CHUNK
        pltpu.sync_copy(idx_hbm.at[pl.ds(2 * base, 2 * CHUNK)], idx_v)
        cg = pltpu.async_copy(t_hbm.at[idx_v], bufpq, semg)
        pltpu.sync_copy(e_hbm.at[pl.ds(base, CHUNK)], bufe)
        for k in range(CHUNK // 16):
            dsc_v[pl.ds(k * 16, 16)] = idx_v[pl.ds(CHUNK + k * 16, 16)] - npv
        cg.wait()

        def _row(i, c2):
            for j in range(D // 16):
                sl = pl.ds(j * 16, 16)
                x = bufpq[i, sl] + bufpq[CHUNK + i, sl] + bufe[i, sl]
                bufe[i, sl] = _gelu16(x)
            return c2

        lax.fori_loop(0, CHUNK, _row, 0)

        for k in range(CHUNK // 16):
            idx16 = dsc_v[pl.ds(k * 16, 16)]
            plsc.addupdate_scatter(deg_v, [idx16], ones16)

        pltpu.sync_copy(bufe, h_sh.at[dsc_v], add=True)
        return carry

    lax.fori_loop(0, CPW, _chunk, 0)
    plsc.subcore_barrier()

    for k in range(ROWS_PER_TILE // CHUNK):
        r = base_row + k * CHUNK
        pltpu.sync_copy(h_sh.at[pl.ds(r, CHUNK)], h_out.at[cid, pl.ds(r, CHUNK)])
    pltpu.sync_copy(deg_v, deg_out.at[cid, sid])


def _post_body(h_ref, deg_ref, nodes_ref, w2_ref, b2_ref, w3a_ref, w3b_ref,
               b3_ref, w4_ref, b4_ref, out_ref):
    h = h_ref[0] + h_ref[1]
    deg = jnp.sum(deg_ref[...], axis=(0, 1))
    agg = (
        jnp.dot(h, w2_ref[...], preferred_element_type=jnp.float32)
        + deg[:, None] * b2_ref[...]
    )
    x = (
        jnp.dot(nodes_ref[...], w3a_ref[...], preferred_element_type=jnp.float32)
        + jnp.dot(agg, w3b_ref[...], preferred_element_type=jnp.float32)
        + b3_ref[...]
    )
    out_ref[...] = (
        jnp.dot(jax.nn.gelu(x), w4_ref[...], preferred_element_type=jnp.float32)
        + b4_ref[...]
    )


def _post(hpart, deg, nodes_p, w2, b2, w3a, w3b, b3, w4, b4):
    blk = 512
    grid = NP // blk
    full = lambda i: (0, 0)
    return pl.pallas_call(
        _post_body,
        grid=(grid,),
        in_specs=[
            pl.BlockSpec((NC, blk, DW), lambda i: (0, i, 0)),
            pl.BlockSpec((NC, NS, blk), lambda i: (0, 0, i)),
            pl.BlockSpec((blk, D), lambda i: (i, 0)),
            pl.BlockSpec((D, D), full),
            pl.BlockSpec((1, D), full),
            pl.BlockSpec((D, D), full),
            pl.BlockSpec((D, D), full),
            pl.BlockSpec((1, D), full),
            pl.BlockSpec((D, D), full),
            pl.BlockSpec((1, D), full),
        ],
        out_specs=pl.BlockSpec((blk, D), lambda i: (i, 0)),
        out_shape=jax.ShapeDtypeStruct((NP, D), jnp.float32),
    )(hpart, deg, nodes_p, w2, b2, w3a, w3b, b3, w4, b4)


def kernel(node_features, edge_indices, edge_features, W1, b1, W2, b2, W3, b3, W4, b4):
    nodes = node_features[0]
    src = edge_indices[0, :, 0]
    dst = edge_indices[0, :, 1]
    ef = edge_features[0]

    pad_e = E_PAD - N_EDGES
    pad_idx = jnp.full((pad_e,), N_NODES, jnp.int32)
    src_p = jnp.concatenate([src, pad_idx])
    dst_p = jnp.concatenate([dst, pad_idx])
    idxcat = jnp.concatenate(
        [src_p.reshape(-1, CHUNK), dst_p.reshape(-1, CHUNK) + NP], axis=1
    ).reshape(-1)
    ef_p = jnp.concatenate([ef, jnp.zeros((pad_e, ED), jnp.float32)])
    nodes_p = jnp.concatenate([nodes, jnp.zeros((NP - N_NODES, D), jnp.float32)])

    W1a, W1b, W1c = W1[:D], W1[D:2 * D], W1[2 * D:]
    W3a, W3b = W3[:D], W3[D:]

    T = _pq(nodes_p, W1a, W1b).reshape(2 * NP, D)
    Epre = _epre(ef_p, W1c, b1.reshape(1, D))
    hpart, deg = _sc_agg(T, Epre, idxcat)
    out_p = _post(hpart, deg, nodes_p, W2, b2.reshape(1, D), W3a, W3b,
                  b3.reshape(1, D), W4, b4.reshape(1, D))
    return out_p[:N_NODES][None]
